# baseline (speedup 1.0000x reference)
import functools

import jax
import jax.numpy as jnp
from jax.experimental import pallas as pl
from jax.experimental.pallas import tpu as pltpu


def _fused_band_kernel(x_ref, xt_ref, xb_ref,
                       s_ref, w1_ref, w2_ref, ws_ref, g_ref,
                       b1_ref, b2_ref, bs_ref, beta_ref,
                       o_ref, *, rows, nb, w, wo, neg_slope):
    b = pl.program_id(1)
    bf = jnp.bfloat16
    cin = x_ref.shape[0]

    xm = jnp.swapaxes(x_ref[...].astype(bf), 0, 1)
    xt = jnp.swapaxes(xt_ref[...].astype(bf), 0, 1)
    xb = jnp.swapaxes(xb_ref[...].astype(bf), 0, 1)
    pieces = [xt[5], xt[6], xt[7]]
    pieces += [xm[l] for l in range(2 * rows)]
    pieces += [xb[0], xb[1]]
    xs = jnp.concatenate(pieces, axis=0)
    sel = jnp.dot(xs, s_ref[...],
                  preferred_element_type=jnp.float32).astype(bf)

    zero_mask = (b > 0).astype(bf)

    def row_eo(l):
        base = (l + 3) * cin
        blk = sel[base:base + cin]
        return blk[:, 0:wo], blk[:, wo:2 * wo]

    zc = None

    def shift_r(v):
        return jnp.concatenate([zc, v[:, 0:wo - 1]], axis=1)

    zc = jnp.zeros((cin, 1), bf)
    cols = []
    for jj in range(-1, rows + 1):
        parts = []
        for kh in range(3):
            l = 2 * jj + kh - 1
            e, o = row_eo(l)
            if l == -1:
                e = e * zero_mask
                o = o * zero_mask
            parts.append(shift_r(o))
            parts.append(e)
            parts.append(o)
        cols.append(jnp.concatenate(parts, axis=0))
    rhs1 = jnp.concatenate(cols, axis=1)
    y1 = jnp.dot(w1_ref[...], rhs1,
                 preferred_element_type=jnp.float32) + b1_ref[...]
    y1 = jnp.where(y1 >= 0.0, y1, y1 * neg_slope)

    cout = y1.shape[0]
    tm = (b > 0).astype(jnp.float32)
    bm = (b < nb - 1).astype(jnp.float32)
    y1 = y1 * jnp.concatenate(
        [jnp.full((1, wo), tm), jnp.ones((1, rows * wo)),
         jnp.full((1, wo), bm)], axis=1)
    y1b = y1.astype(bf)

    zc2 = jnp.zeros((cout, 1), bf)
    cols2 = []
    for r in range(rows):
        parts = []
        for kh in range(3):
            g = y1b[:, (r + kh) * wo:(r + kh + 1) * wo]
            parts.append(jnp.concatenate([zc2, g[:, 0:wo - 1]], axis=1))
            parts.append(g)
            parts.append(jnp.concatenate([g[:, 1:wo], zc2], axis=1))
        cols2.append(jnp.concatenate(parts, axis=0))
    rhs2 = jnp.concatenate(cols2, axis=1)
    y2 = jnp.dot(w2_ref[...], rhs2,
                 preferred_element_type=jnp.float32) + b2_ref[...]

    srhs = jnp.concatenate(
        [row_eo(2 * r)[0] for r in range(rows)], axis=1)
    ident = jnp.dot(ws_ref[...], srhs,
                    preferred_element_type=jnp.float32) + bs_ref[...]

    ysq = (y2 * y2).astype(bf)
    norm = jnp.dot(g_ref[...], ysq,
                   preferred_element_type=jnp.float32) + beta_ref[...]
    res = y2 * jax.lax.rsqrt(norm) + ident
    o_ref[...] = res.reshape(cout, rows, wo)


def _pick_band(ho):
    for r in (16, 8, 4):
        if ho % r == 0:
            return r
    raise NotImplementedError("output height must be divisible by 4")


def _forward(x, smat, w1l, w2l, wsl, gl, b1, b2, bs, beta):
    n, cin, h, w = x.shape
    cout = w1l.shape[0]
    ho = (h - 1) // 2 + 1
    wo = (w - 1) // 2 + 1
    rows = _pick_band(ho)
    nb = ho // rows

    xf = x.reshape(n * cin, h, w)

    cparams = pltpu.CompilerParams(
        dimension_semantics=("parallel", "parallel"),
        vmem_limit_bytes=64 * 1024 * 1024,
    )

    def const_spec(shape):
        return pl.BlockSpec(shape, lambda i, b: (0,) * len(shape))

    u = 2 * rows // 8

    outf = pl.pallas_call(
        functools.partial(_fused_band_kernel, rows=rows, nb=nb, w=w, wo=wo,
                          neg_slope=0.01),
        out_shape=jax.ShapeDtypeStruct((n * cout, ho, wo), jnp.float32),
        grid=(n, nb),
        in_specs=[
            pl.BlockSpec((cin, 2 * rows, w), lambda i, b: (i, b, 0)),
            pl.BlockSpec((cin, 8, w),
                         lambda i, b: (i, jnp.maximum(u * b - 1, 0), 0)),
            pl.BlockSpec((cin, 8, w),
                         lambda i, b: (i, jnp.minimum(u * (b + 1), h // 8 - 1), 0)),
            const_spec((w, 2 * wo)),
            const_spec((cout, 9 * cin)),
            const_spec((cout, 9 * cout)),
            const_spec((cout, cin)),
            const_spec((cout, cout)),
            const_spec((cout, 1)), const_spec((cout, 1)),
            const_spec((cout, 1)), const_spec((cout, 1)),
        ],
        out_specs=pl.BlockSpec((cout, rows, wo), lambda i, b: (i, b, 0)),
        compiler_params=cparams,
    )(xf, xf, xf, smat, w1l, w2l, wsl, gl, b1, b2, bs, beta)

    return outf.reshape(n, cout, ho, wo)


def kernel(x, mask1, mask2, w1, b1, w2, b2, ws, bs, gamma, beta):
    del mask1, mask2
    n, cin, h, w = x.shape
    cout = w1.shape[1]
    wo = (w - 1) // 2 + 1

    jj = jnp.arange(w)[:, None]
    ii = jnp.arange(wo)[None, :]
    smat = jnp.concatenate(
        [(jj == 2 * ii), (jj == 2 * ii + 1)], axis=1).astype(jnp.bfloat16)

    w1l = jnp.transpose(w1, (1, 0, 2)).reshape(cout, 9 * cin).astype(jnp.bfloat16)
    w2l = jnp.transpose(w2, (1, 0, 2)).reshape(cout, 9 * cout).astype(jnp.bfloat16)
    wsl = ws.astype(jnp.bfloat16)
    gl = gamma.astype(jnp.bfloat16)

    args = (smat, w1l, w2l, wsl, gl, b1, b2, bs, beta)
    devs = jax.devices()
    if len(devs) > 1 and n % 2 == 0:
        mesh = jax.sharding.Mesh(devs[:2], ("d",))
        pd = jax.sharding.PartitionSpec("d")
        pr = jax.sharding.PartitionSpec()
        fwd = jax.shard_map(
            _forward, mesh=mesh,
            in_specs=(pd,) + (pr,) * len(args), out_specs=pd,
            check_vma=False)
        return fwd(x, *args)
    return _forward(x, *args)


# speedup vs baseline: 3.8357x; 3.8357x over previous
import functools

import jax
import jax.numpy as jnp
from jax.experimental import pallas as pl
from jax.experimental.pallas import tpu as pltpu


def _fused_band_kernel(x_ref, xt_ref, xb_ref,
                       s_ref, w1_ref, w2_ref, ws_ref, g_ref,
                       b1_ref, b2_ref, bs_ref, beta_ref,
                       o_ref, *, rows, nb, w, wo, neg_slope):
    b = pl.program_id(1)
    bf = jnp.bfloat16
    cin = x_ref.shape[0]

    xm = jnp.swapaxes(x_ref[...].astype(bf), 0, 1)
    xt = jnp.swapaxes(xt_ref[...][:, 5:8, :].astype(bf), 0, 1)
    xb = jnp.swapaxes(xb_ref[...][:, 0:2, :].astype(bf), 0, 1)
    pieces = [xt[0], xt[1], xt[2]]
    pieces += [xm[l] for l in range(2 * rows)]
    pieces += [xb[0], xb[1]]
    xs = jnp.concatenate(pieces, axis=0)
    sel = jnp.dot(xs, s_ref[...],
                  preferred_element_type=jnp.float32).astype(bf)

    zero_mask = (b > 0).astype(bf)

    def row_eo(l):
        base = (l + 3) * cin
        blk = sel[base:base + cin]
        return blk[:, 0:wo], blk[:, wo:2 * wo]

    zc = None

    def shift_r(v):
        return jnp.concatenate([zc, v[:, 0:wo - 1]], axis=1)

    zc = jnp.zeros((cin, 1), bf)
    cols = []
    for jj in range(-1, rows + 1):
        parts = []
        for kh in range(3):
            l = 2 * jj + kh - 1
            e, o = row_eo(l)
            if l == -1:
                e = e * zero_mask
                o = o * zero_mask
            parts.append(shift_r(o))
            parts.append(e)
            parts.append(o)
        cols.append(jnp.concatenate(parts, axis=0))
    rhs1 = jnp.concatenate(cols, axis=1)
    y1 = jnp.dot(w1_ref[...], rhs1,
                 preferred_element_type=jnp.float32) + b1_ref[...]
    y1 = jnp.where(y1 >= 0.0, y1, y1 * neg_slope)

    cout = y1.shape[0]
    y1b = y1.astype(bf)
    tm = (b > 0).astype(bf)
    bm = (b < nb - 1).astype(bf)

    zc2 = jnp.zeros((cout, 1), bf)
    cols2 = []
    for r in range(rows):
        parts = []
        for kh in range(3):
            g = y1b[:, (r + kh) * wo:(r + kh + 1) * wo]
            if r + kh == 0:
                g = g * tm
            elif r + kh == rows + 1:
                g = g * bm
            parts.append(jnp.concatenate([zc2, g[:, 0:wo - 1]], axis=1))
            parts.append(g)
            parts.append(jnp.concatenate([g[:, 1:wo], zc2], axis=1))
        cols2.append(jnp.concatenate(parts, axis=0))
    rhs2 = jnp.concatenate(cols2, axis=1)
    y2 = jnp.dot(w2_ref[...], rhs2,
                 preferred_element_type=jnp.float32) + b2_ref[...]

    srhs = jnp.concatenate(
        [row_eo(2 * r)[0] for r in range(rows)], axis=1)
    ident = jnp.dot(ws_ref[...], srhs,
                    preferred_element_type=jnp.float32) + bs_ref[...]

    ysq = (y2 * y2).astype(bf)
    norm = jnp.dot(g_ref[...], ysq,
                   preferred_element_type=jnp.float32) + beta_ref[...]
    res = y2 * jax.lax.rsqrt(norm) + ident
    o_ref[...] = res.reshape(cout, rows, wo)


def _pick_band(ho):
    for r in (16, 8, 4):
        if ho % r == 0:
            return r
    raise NotImplementedError("output height must be divisible by 4")


def _forward(x, smat, w1l, w2l, wsl, gl, b1, b2, bs, beta):
    n, cin, h, w = x.shape
    cout = w1l.shape[0]
    ho = (h - 1) // 2 + 1
    wo = (w - 1) // 2 + 1
    rows = _pick_band(ho)
    nb = ho // rows

    xf = x.reshape(n * cin, h, w)

    cparams = pltpu.CompilerParams(
        dimension_semantics=("parallel", "parallel"),
        vmem_limit_bytes=64 * 1024 * 1024,
    )

    def const_spec(shape):
        return pl.BlockSpec(shape, lambda i, b: (0,) * len(shape))

    u = 2 * rows // 8

    outf = pl.pallas_call(
        functools.partial(_fused_band_kernel, rows=rows, nb=nb, w=w, wo=wo,
                          neg_slope=0.01),
        out_shape=jax.ShapeDtypeStruct((n * cout, ho, wo), jnp.float32),
        grid=(n, nb),
        in_specs=[
            pl.BlockSpec((cin, 2 * rows, w), lambda i, b: (i, b, 0)),
            pl.BlockSpec((cin, 8, w),
                         lambda i, b: (i, jnp.maximum(u * b - 1, 0), 0)),
            pl.BlockSpec((cin, 8, w),
                         lambda i, b: (i, jnp.minimum(u * (b + 1), h // 8 - 1), 0)),
            const_spec((w, 2 * wo)),
            const_spec((cout, 9 * cin)),
            const_spec((cout, 9 * cout)),
            const_spec((cout, cin)),
            const_spec((cout, cout)),
            const_spec((cout, 1)), const_spec((cout, 1)),
            const_spec((cout, 1)), const_spec((cout, 1)),
        ],
        out_specs=pl.BlockSpec((cout, rows, wo), lambda i, b: (i, b, 0)),
        compiler_params=cparams,
    )(xf, xf, xf, smat, w1l, w2l, wsl, gl, b1, b2, bs, beta)

    return outf.reshape(n, cout, ho, wo)


def kernel(x, mask1, mask2, w1, b1, w2, b2, ws, bs, gamma, beta):
    del mask1, mask2
    n, cin, h, w = x.shape
    cout = w1.shape[1]
    wo = (w - 1) // 2 + 1

    jj = jnp.arange(w)[:, None]
    ii = jnp.arange(wo)[None, :]
    smat = jnp.concatenate(
        [(jj == 2 * ii), (jj == 2 * ii + 1)], axis=1).astype(jnp.bfloat16)

    w1l = jnp.transpose(w1, (1, 0, 2)).reshape(cout, 9 * cin).astype(jnp.bfloat16)
    w2l = jnp.transpose(w2, (1, 0, 2)).reshape(cout, 9 * cout).astype(jnp.bfloat16)
    wsl = ws.astype(jnp.bfloat16)
    gl = gamma.astype(jnp.bfloat16)

    return _forward(x, smat, w1l, w2l, wsl, gl, b1, b2, bs, beta)


# speedup vs baseline: 4.1649x; 1.0858x over previous
import functools

import jax
import jax.numpy as jnp
from jax.experimental import pallas as pl
from jax.experimental.pallas import tpu as pltpu


def _fused_band_kernel(x_ref, xt_ref, xb_ref, s_ref, wc_ref, bc_ref,
                       o_ref, *, rows, nb, w, wo, neg_slope):
    b = pl.program_id(1)
    bf = jnp.bfloat16
    cin = x_ref.shape[0]
    cout = wc_ref.shape[0]
    k1 = 9 * cin
    w1_ref_v = wc_ref[:, 0:k1]
    ws_ref_v = wc_ref[:, k1:k1 + cin]
    gpad = ((cout + 127) // 128) * 128
    g_ref_v = wc_ref[:, k1 + cin:k1 + cin + cout]
    w2_ref_v = wc_ref[:, k1 + cin + gpad:k1 + cin + gpad + 9 * cout]
    b1_c = bc_ref[:, 0:1]
    b2_c = bc_ref[:, 1:2]
    bs_c = bc_ref[:, 2:3]
    beta_c = bc_ref[:, 3:4]

    xm = jnp.swapaxes(x_ref[...].astype(bf), 0, 1)
    xt = jnp.swapaxes(xt_ref[...].astype(bf), 0, 1)
    xb = jnp.swapaxes(xb_ref[...].astype(bf), 0, 1)
    pieces = [xt[5], xt[6], xt[7]]
    pieces += [xm[l] for l in range(2 * rows)]
    pieces += [xb[0], xb[1]]
    xs = jnp.concatenate(pieces, axis=0)
    sel = jnp.dot(xs, s_ref[...],
                  preferred_element_type=jnp.float32).astype(bf)

    zero_mask = (b > 0).astype(bf)

    def row_eo(l):
        base = (l + 3) * cin
        blk = sel[base:base + cin]
        return blk[:, 0:wo], blk[:, wo:2 * wo]

    zc = None

    def shift_r(v):
        return jnp.concatenate([zc, v[:, 0:wo - 1]], axis=1)

    zc = jnp.zeros((cin, 1), bf)
    cols = []
    for jj in range(-1, rows + 1):
        parts = []
        for kh in range(3):
            l = 2 * jj + kh - 1
            e, o = row_eo(l)
            if l == -1:
                e = e * zero_mask
                o = o * zero_mask
            parts.append(shift_r(o))
            parts.append(e)
            parts.append(o)
        cols.append(jnp.concatenate(parts, axis=0))
    rhs1 = jnp.concatenate(cols, axis=1)
    y1 = jnp.dot(w1_ref_v, rhs1,
                 preferred_element_type=jnp.float32) + b1_c
    y1 = jnp.where(y1 >= 0.0, y1, y1 * neg_slope)

    tm = (b > 0).astype(jnp.float32)
    bm = (b < nb - 1).astype(jnp.float32)
    y1 = y1 * jnp.concatenate(
        [jnp.full((1, wo), tm), jnp.ones((1, rows * wo)),
         jnp.full((1, wo), bm)], axis=1)
    y1b = y1.astype(bf)

    zc2 = jnp.zeros((cout, 1), bf)
    cols2 = []
    for r in range(rows):
        parts = []
        for kh in range(3):
            g = y1b[:, (r + kh) * wo:(r + kh + 1) * wo]
            parts.append(jnp.concatenate([zc2, g[:, 0:wo - 1]], axis=1))
            parts.append(g)
            parts.append(jnp.concatenate([g[:, 1:wo], zc2], axis=1))
        cols2.append(jnp.concatenate(parts, axis=0))
    rhs2 = jnp.concatenate(cols2, axis=1)
    y2 = jnp.dot(w2_ref_v, rhs2,
                 preferred_element_type=jnp.float32) + b2_c

    srhs = jnp.concatenate(
        [row_eo(2 * r)[0] for r in range(rows)], axis=1)
    ident = jnp.dot(ws_ref_v, srhs,
                    preferred_element_type=jnp.float32) + bs_c

    ysq = (y2 * y2).astype(bf)
    norm = jnp.dot(g_ref_v, ysq,
                   preferred_element_type=jnp.float32) + beta_c
    res = y2 * jax.lax.rsqrt(norm) + ident
    o_ref[...] = res.reshape(cout, rows, wo)


def _pick_band(ho):
    for r in (16, 8, 4):
        if ho % r == 0:
            return r
    raise NotImplementedError("output height must be divisible by 4")


def _forward(x, smat, wcat, bcat):
    n, cin, h, w = x.shape
    cout = wcat.shape[0]
    ho = (h - 1) // 2 + 1
    wo = (w - 1) // 2 + 1
    rows = _pick_band(ho)
    nb = ho // rows

    xf = x.reshape(n * cin, h, w)

    cparams = pltpu.CompilerParams(
        dimension_semantics=("parallel", "parallel"),
        vmem_limit_bytes=64 * 1024 * 1024,
    )

    def const_spec(shape):
        return pl.BlockSpec(shape, lambda i, b: (0,) * len(shape))

    u = 2 * rows // 8

    outf = pl.pallas_call(
        functools.partial(_fused_band_kernel, rows=rows, nb=nb, w=w, wo=wo,
                          neg_slope=0.01),
        out_shape=jax.ShapeDtypeStruct((n * cout, ho, wo), jnp.float32),
        grid=(n, nb),
        in_specs=[
            pl.BlockSpec((cin, 2 * rows, w), lambda i, b: (i, b, 0)),
            pl.BlockSpec((cin, 8, w),
                         lambda i, b: (i, jnp.maximum(u * b - 1, 0), 0)),
            pl.BlockSpec((cin, 8, w),
                         lambda i, b: (i, jnp.minimum(u * (b + 1), h // 8 - 1), 0)),
            const_spec((w, 2 * wo)),
            const_spec(wcat.shape),
            const_spec(bcat.shape),
        ],
        out_specs=pl.BlockSpec((cout, rows, wo), lambda i, b: (i, b, 0)),
        compiler_params=cparams,
    )(xf, xf, xf, smat, wcat, bcat)

    return outf.reshape(n, cout, ho, wo)


def kernel(x, mask1, mask2, w1, b1, w2, b2, ws, bs, gamma, beta):
    del mask1, mask2
    n, cin, h, w = x.shape
    cout = w1.shape[1]
    wo = (w - 1) // 2 + 1

    jj = jnp.arange(w)[:, None]
    ii = jnp.arange(wo)[None, :]
    smat = jnp.concatenate(
        [(jj == 2 * ii), (jj == 2 * ii + 1)], axis=1).astype(jnp.bfloat16)

    w1l = jnp.transpose(w1, (1, 0, 2)).reshape(cout, 9 * cin).astype(jnp.bfloat16)
    w2l = jnp.transpose(w2, (1, 0, 2)).reshape(cout, 9 * cout).astype(jnp.bfloat16)
    wsl = ws.astype(jnp.bfloat16)
    gpad = ((cout + 127) // 128) * 128
    glp = jnp.zeros((cout, gpad), jnp.bfloat16).at[:, :cout].set(
        gamma.astype(jnp.bfloat16))
    wcat = jnp.concatenate([w1l, wsl, glp, w2l], axis=1)
    bcat = jnp.concatenate([b1, b2, bs, beta], axis=1)

    return _forward(x, smat, wcat, bcat)
